# baseline (speedup 1.0000x reference)
import functools

import jax
import jax.numpy as jnp
from jax.experimental import pallas as pl
from jax.experimental.pallas import tpu as pltpu

_LANE = 128
_SUBLANE = 8


def _round_up(x, m):
    return (x + m - 1) // m * m


def _mlp_kernel(x_ref, w0_ref, b0_ref, w1_ref, b1_ref, w2_ref, b2_ref, o_ref,
                *, chunk_m):
    block_m = x_ref.shape[0]
    for c in range(block_m // chunk_m):
        rows = pl.ds(c * chunk_m, chunk_m)
        z1 = jnp.dot(x_ref[rows, :], w0_ref[...],
                     preferred_element_type=jnp.float32) + b0_ref[...]
        h1 = jnp.maximum(z1, 0.0)
        z2 = jnp.dot(h1, w1_ref[...],
                     preferred_element_type=jnp.float32) + b1_ref[...]
        h2 = jnp.maximum(z2, 0.0)
        z3 = jnp.dot(h2, w2_ref[...],
                     preferred_element_type=jnp.float32) + b2_ref[...]
        o_ref[rows, :] = z3.astype(o_ref.dtype)


def kernel(x, w0, b0, w1, b1, w2, b2, *, block_m=1024, chunk_m=1024):
    M, K = x.shape
    ws = [w0, w1, w2]
    bs = [b0, b1, b2]
    dims = [K] + [w.shape[1] for w in ws]
    pad_dims = [_round_up(d, _LANE) for d in dims]

    x_p = jnp.pad(x, ((0, 0), (0, pad_dims[0] - dims[0])))
    flat_params = []
    for i, (w, b) in enumerate(zip(ws, bs)):
        kin, kout = w.shape
        w_p = jnp.pad(w, ((0, pad_dims[i] - kin),
                          (0, pad_dims[i + 1] - kout)))
        b_p = jnp.pad(b, (0, pad_dims[i + 1] - kout)).reshape(1, pad_dims[i + 1])
        flat_params.extend((w_p, b_p))

    block_m = min(_round_up(M, _SUBLANE), block_m)
    chunk_m = min(chunk_m, block_m)
    if block_m % chunk_m:
        chunk_m = block_m
    m_pad = _round_up(M, block_m)
    if m_pad != M:
        x_p = jnp.pad(x_p, ((0, m_pad - M), (0, 0)))
    grid_m = m_pad // block_m

    steps_per_core = max(grid_m // 2, 1)
    n_cores = grid_m // steps_per_core

    def x_map(i, j):
        return (i * steps_per_core + j, 0)

    in_specs = [pl.BlockSpec((block_m, pad_dims[0]), x_map)]
    for p in flat_params:
        in_specs.append(pl.BlockSpec(p.shape, lambda i, j: (0, 0)))

    flops = 2 * M * sum(dims[i] * dims[i + 1] for i in range(3))
    bytes_accessed = (
        x_p.size * x_p.dtype.itemsize
        + sum(p.size * p.dtype.itemsize for p in flat_params)
        + M * dims[-1] * 4
    )

    out_p = pl.pallas_call(
        functools.partial(_mlp_kernel, chunk_m=chunk_m),
        out_shape=jax.ShapeDtypeStruct((m_pad, pad_dims[-1]), x.dtype),
        grid=(n_cores, steps_per_core),
        in_specs=in_specs,
        out_specs=pl.BlockSpec((block_m, pad_dims[-1]), x_map),
        compiler_params=pltpu.CompilerParams(
            dimension_semantics=("parallel", "arbitrary"),
        ),
        cost_estimate=pl.CostEstimate(
            flops=flops, transcendentals=0, bytes_accessed=bytes_accessed),
    )(x_p, *flat_params)

    return out_p[:M, : dims[-1]]
